# baseline (speedup 1.0000x reference)
import functools

import numpy as np
import jax
import jax.numpy as jnp
from jax import lax
from jax.experimental import pallas as pl
from jax.experimental.pallas import tpu as pltpu
from jax.experimental.pallas import tpu_sc as plsc

_INT_MAX = 2**31 - 1
_START = (0, 0, 0)
_END = (4, _INT_MAX, 4096)
_STEP = (1, 2, 2)

_LANES = 16


def _static_indices(shape):
    idx = []
    for i in range(len(shape)):
        e = shape[i] if _END[i] == _INT_MAX else _END[i]
        n = max(0, -((_START[i] - e) // _STEP[i]))
        ids = _START[i] + _STEP[i] * np.arange(n, dtype=np.int64)
        ids = np.minimum(ids, e - 1)
        idx.append(ids)
    return idx


@functools.lru_cache(maxsize=None)
def _build(S2, R, n2, C, c_start, c_step, c_end):
    info = plsc.get_sparse_core_info()
    nw = info.num_cores * info.num_subcores
    rows_per_tile = R // nw
    nchunk = rows_per_tile // C
    groups = n2 // _LANES
    c_clamp_val = c_end - 1

    mesh = plsc.VectorSubcoreMesh(core_axis_name="c", subcore_axis_name="s")

    def body(x_hbm, rowidx_hbm, out_hbm,
             ridx_v, in_v0, in_v1, out_v0, out_v1,
             sem_in0, sem_in1, sem_out0, sem_out1):
        wid = lax.axis_index("s") * info.num_cores + lax.axis_index("c")
        base = wid * rows_per_tile
        pltpu.sync_copy(rowidx_hbm.at[pl.ds(base, rows_per_tile)], ridx_v)

        cstep_iota = c_step * lax.iota(jnp.int32, _LANES)
        c_clamp = jnp.full((_LANES,), c_clamp_val, dtype=jnp.int32)
        in_v = (in_v0, in_v1)
        out_v = (out_v0, out_v1)
        sem_in = (sem_in0, sem_in1)
        sem_out = (sem_out0, sem_out1)
        rvecs = [jnp.full((_LANES,), r, dtype=jnp.int32) for r in range(C)]

        def start_gather(b, g):
            pltpu.async_copy(
                x_hbm.at[pl.ds(base + g * C, C)], in_v[b], sem_in[b])

        def compute(b):
            def j_body(j2, carry):
                for u in range(2):
                    j = j2 * 2 + u
                    cvec = jnp.minimum(
                        jnp.full((_LANES,), c_start + c_step * _LANES * j,
                                 dtype=jnp.int32) + cstep_iota,
                        c_clamp)
                    col0 = j * _LANES
                    vs = [plsc.load_gather(in_v[b], [rvecs[r], cvec])
                          for r in range(C)]
                    for r in range(C):
                        out_v[b][r, pl.ds(col0, _LANES)] = vs[r]
                return carry

            lax.fori_loop(0, groups // 2, j_body, 0)

        start_gather(0, 0)
        start_gather(1, 1)

        def pair_body(g2, carry):
            for b in range(2):
                g = g2 * 2 + b
                row0 = base + g * C

                @pl.when(g2 > 0)
                def _():
                    pltpu.make_async_copy(
                        out_v[b], out_hbm.at[pl.ds(row0, C)],
                        sem_out[b]).wait()

                pltpu.make_async_copy(
                    x_hbm.at[pl.ds(base + g * C, C)], in_v[b],
                    sem_in[b]).wait()
                compute(b)
                pltpu.async_copy(
                    out_v[b], out_hbm.at[pl.ds(row0, C)],
                    sem_out[b])

                @pl.when(g + 2 < nchunk)
                def _():
                    start_gather(b, g + 2)
            return carry

        lax.fori_loop(0, nchunk // 2, pair_body, 0)

        for b in range(2):
            g = nchunk - 2 + b
            row0 = base + g * C
            pltpu.make_async_copy(
                out_v[b], out_hbm.at[pl.ds(row0, C)],
                sem_out[b]).wait()

    return pl.kernel(
        body,
        out_type=jax.ShapeDtypeStruct((R, n2), jnp.float32),
        mesh=mesh,
        compiler_params=pltpu.CompilerParams(
            use_tc_tiling_on_sc=True, needs_layout_passes=False,
            disable_bounds_checks=True),
        scratch_types=[
            pltpu.VMEM((rows_per_tile,), jnp.int32),
            pltpu.VMEM((C, S2), jnp.float32),
            pltpu.VMEM((C, S2), jnp.float32),
            pltpu.VMEM((C, n2), jnp.float32),
            pltpu.VMEM((C, n2), jnp.float32),
            pltpu.SemaphoreType.DMA,
            pltpu.SemaphoreType.DMA,
            pltpu.SemaphoreType.DMA,
            pltpu.SemaphoreType.DMA,
        ],
    )


def kernel(input, start, end, step):
    S0, S1, S2 = input.shape
    idx0, idx1, idx2 = _static_indices(input.shape)
    n0, n1, n2 = len(idx0), len(idx1), len(idx2)
    R = n0 * n1
    row_idx = (idx0[:, None] * S1 + idx1[None, :]).reshape(-1).astype(np.int32)
    x_flat = input.reshape(S0 * S1, S2)
    c_end = S2 if _END[2] == _INT_MAX else _END[2]
    run = _build(S2, R, n2, 8, _START[2], _STEP[2], c_end)
    out = run(x_flat, jnp.asarray(row_idx))
    return out.reshape(n0, n1, n2)


# speedup vs baseline: 1.0067x; 1.0067x over previous
import functools

import numpy as np
import jax
import jax.numpy as jnp
from jax import lax
from jax.experimental import pallas as pl
from jax.experimental.pallas import tpu as pltpu
from jax.experimental.pallas import tpu_sc as plsc

_INT_MAX = 2**31 - 1
_START = (0, 0, 0)
_END = (4, _INT_MAX, 4096)
_STEP = (1, 2, 2)

_LANES = 16


def _static_indices(shape):
    idx = []
    for i in range(len(shape)):
        e = shape[i] if _END[i] == _INT_MAX else _END[i]
        n = max(0, -((_START[i] - e) // _STEP[i]))
        ids = _START[i] + _STEP[i] * np.arange(n, dtype=np.int64)
        ids = np.minimum(ids, e - 1)
        idx.append(ids)
    return idx


@functools.lru_cache(maxsize=None)
def _build(S2, R, n2, C, c_start, c_step, c_end):
    info = plsc.get_sparse_core_info()
    nw = info.num_cores * info.num_subcores
    rows_per_tile = R // nw
    nchunk = rows_per_tile // C
    groups = n2 // _LANES
    c_clamp_val = c_end - 1

    mesh = plsc.VectorSubcoreMesh(core_axis_name="c", subcore_axis_name="s")

    def body(x_hbm, rowidx_hbm, out_hbm,
             ridx_v, in_v0, in_v1, out_v0, out_v1,
             sem_in0, sem_in1, sem_out0, sem_out1):
        wid = lax.axis_index("s") * info.num_cores + lax.axis_index("c")
        base = wid * rows_per_tile
        pltpu.sync_copy(rowidx_hbm.at[pl.ds(base, rows_per_tile)], ridx_v)

        cstep_iota = c_step * lax.iota(jnp.int32, _LANES)
        c_clamp = jnp.full((_LANES,), c_clamp_val, dtype=jnp.int32)
        in_v = (in_v0, in_v1)
        out_v = (out_v0, out_v1)
        sem_in = (sem_in0, sem_in1)
        sem_out = (sem_out0, sem_out1)
        rvecs = [jnp.full((_LANES,), r, dtype=jnp.int32) for r in range(C)]

        def start_gather(b, g):
            pltpu.async_copy(
                x_hbm.at[ridx_v.at[pl.ds(g * C, C)]], in_v[b], sem_in[b])

        def compute(b):
            def j_body(j2, carry):
                for u in range(2):
                    j = j2 * 2 + u
                    cvec = jnp.minimum(
                        jnp.full((_LANES,), c_start + c_step * _LANES * j,
                                 dtype=jnp.int32) + cstep_iota,
                        c_clamp)
                    col0 = j * _LANES
                    vs = [plsc.load_gather(in_v[b], [rvecs[r], cvec])
                          for r in range(C)]
                    for r in range(C):
                        out_v[b][r, pl.ds(col0, _LANES)] = vs[r]
                return carry

            lax.fori_loop(0, groups // 2, j_body, 0)

        start_gather(0, 0)
        start_gather(1, 1)

        def pair_body(g2, carry):
            for b in range(2):
                g = g2 * 2 + b
                row0 = base + g * C

                @pl.when(g2 > 0)
                def _():
                    pltpu.make_async_copy(
                        out_v[b], out_hbm.at[pl.ds(row0, C)],
                        sem_out[b]).wait()

                pltpu.make_async_copy(
                    x_hbm.at[ridx_v.at[pl.ds(g * C, C)]], in_v[b],
                    sem_in[b]).wait()
                compute(b)
                pltpu.async_copy(
                    out_v[b], out_hbm.at[pl.ds(row0, C)],
                    sem_out[b])

                @pl.when(g + 2 < nchunk)
                def _():
                    start_gather(b, g + 2)
            return carry

        lax.fori_loop(0, nchunk // 2, pair_body, 0)

        for b in range(2):
            g = nchunk - 2 + b
            row0 = base + g * C
            pltpu.make_async_copy(
                out_v[b], out_hbm.at[pl.ds(row0, C)],
                sem_out[b]).wait()

    return pl.kernel(
        body,
        out_type=jax.ShapeDtypeStruct((R, n2), jnp.float32),
        mesh=mesh,
        compiler_params=pltpu.CompilerParams(
            use_tc_tiling_on_sc=True, needs_layout_passes=False,
            disable_bounds_checks=True),
        scratch_types=[
            pltpu.VMEM((rows_per_tile,), jnp.int32),
            pltpu.VMEM((C, S2), jnp.float32),
            pltpu.VMEM((C, S2), jnp.float32),
            pltpu.VMEM((C, n2), jnp.float32),
            pltpu.VMEM((C, n2), jnp.float32),
            pltpu.SemaphoreType.DMA,
            pltpu.SemaphoreType.DMA,
            pltpu.SemaphoreType.DMA,
            pltpu.SemaphoreType.DMA,
        ],
    )


def kernel(input, start, end, step):
    S0, S1, S2 = input.shape
    idx0, idx1, idx2 = _static_indices(input.shape)
    n0, n1, n2 = len(idx0), len(idx1), len(idx2)
    R = n0 * n1
    row_idx = (idx0[:, None] * S1 + idx1[None, :]).reshape(-1).astype(np.int32)
    x_flat = input.reshape(S0 * S1, S2)
    c_end = S2 if _END[2] == _INT_MAX else _END[2]
    run = _build(S2, R, n2, 8, _START[2], _STEP[2], c_end)
    out = run(x_flat, jnp.asarray(row_idx))
    return out.reshape(n0, n1, n2)
